# baseline (speedup 1.0000x reference)
import functools

import jax
import jax.numpy as jnp
from jax import lax
from jax.experimental import pallas as pl
from jax.experimental.pallas import tpu as pltpu
from jax.experimental.pallas import tpu_sc as plsc

ROWS, COLS = 16384, 128
NUM_CORES, NUM_SUBCORES = 2, 16
NUM_WORKERS = NUM_CORES * NUM_SUBCORES
ROWS_PER_WORKER = ROWS // NUM_WORKERS
LANES = 16
NSL = COLS // LANES
BLK = 64

_MESH = plsc.VectorSubcoreMesh(
    core_axis_name="c", subcore_axis_name="s",
    num_cores=NUM_CORES, num_subcores=NUM_SUBCORES,
)


CHUNK = 64
NCHUNK = ROWS_PER_WORKER // CHUNK


def _reduce_body(x_hbm, part_hbm, rows_v, acc_v, comb_v, shared, sem0, sem1):
    cid = lax.axis_index("c")
    sid = lax.axis_index("s")
    wid = cid * NUM_SUBCORES + sid
    base = wid * ROWS_PER_WORKER

    sems = (sem0, sem1)
    copies = [None, None]
    copies[0] = pltpu.make_async_copy(
        x_hbm.at[pl.ds(base, CHUNK)], rows_v.at[0], sems[0]
    )
    copies[0].start()

    accs = tuple(jnp.zeros((LANES,), jnp.float32) for _ in range(NSL))
    for k in range(NCHUNK):
        buf = k % 2
        if k + 1 < NCHUNK:
            copies[1 - buf] = pltpu.make_async_copy(
                x_hbm.at[pl.ds(base + (k + 1) * CHUNK, CHUNK)],
                rows_v.at[1 - buf],
                sems[1 - buf],
            )
            copies[1 - buf].start()
        copies[buf].wait()

        @plsc.parallel_loop(0, CHUNK, unroll=4, carry=accs)
        def accs(r, accs, buf=buf):
            return tuple(
                a + rows_v[buf, r, pl.ds(LANES * j, LANES)]
                for j, a in enumerate(accs)
            )

    for j, a in enumerate(accs):
        acc_v[pl.ds(LANES * j, LANES)] = a

    pltpu.sync_copy(acc_v, shared.at[sid])
    plsc.subcore_barrier()

    @pl.when(sid == 0)
    def _():
        pltpu.sync_copy(shared, comb_v)
        for j in range(NSL):
            t = comb_v[0, pl.ds(LANES * j, LANES)]
            for rr in range(1, NUM_SUBCORES):
                t = t + comb_v[rr, pl.ds(LANES * j, LANES)]
            acc_v[pl.ds(LANES * j, LANES)] = t
        pltpu.sync_copy(acc_v, part_hbm.at[cid])


def _write_body(part_hbm, out_hbm, part_v, block_v, out_sem):
    cid = lax.axis_index("c")
    sid = lax.axis_index("s")
    wid = cid * NUM_SUBCORES + sid
    base = wid * ROWS_PER_WORKER

    pltpu.sync_copy(part_hbm, part_v)

    iota = lax.broadcasted_iota(jnp.int32, (LANES,), 0)
    cands = []
    for j in range(NSL):
        s_j = part_v[0, pl.ds(LANES * j, LANES)] + part_v[1, pl.ds(LANES * j, LANES)]
        ffs = plsc.all_reduce_ffs(s_j >= 0.0)
        cands.append(jnp.where(ffs < LANES, ffs + LANES * j, COLS))
    idx = functools.reduce(jnp.minimum, cands)
    idx = jnp.where(idx >= COLS, 0, idx)

    vrow = [
        jnp.where(iota + LANES * j == idx, 1.0, -1.0).astype(jnp.float32)
        for j in range(NSL)
    ]

    @plsc.parallel_loop(0, BLK, unroll=4)
    def _(r):
        for j in range(NSL):
            block_v[r, pl.ds(LANES * j, LANES)] = vrow[j]

    copies = [
        pltpu.make_async_copy(
            block_v, out_hbm.at[pl.ds(base + b * BLK, BLK)], out_sem
        )
        for b in range(ROWS_PER_WORKER // BLK)
    ]
    for c in copies:
        c.start()
    for c in copies:
        c.wait()


_PARAMS = pltpu.CompilerParams(needs_layout_passes=False)

_reduce = pl.kernel(
    _reduce_body,
    out_type=jax.ShapeDtypeStruct((NUM_CORES, COLS), jnp.float32),
    mesh=_MESH,
    compiler_params=_PARAMS,
    scratch_types=[
        pltpu.VMEM((2, CHUNK, COLS), jnp.float32),
        pltpu.VMEM((COLS,), jnp.float32),
        pltpu.VMEM((NUM_SUBCORES, COLS), jnp.float32),
        pltpu.VMEM_SHARED((NUM_SUBCORES, COLS), jnp.float32),
        pltpu.SemaphoreType.DMA,
        pltpu.SemaphoreType.DMA,
    ],
)

_write = pl.kernel(
    _write_body,
    out_type=jax.ShapeDtypeStruct((ROWS, COLS), jnp.float32),
    mesh=_MESH,
    compiler_params=_PARAMS,
    scratch_types=[
        pltpu.VMEM((NUM_CORES, COLS), jnp.float32),
        pltpu.VMEM((BLK, COLS), jnp.float32),
        pltpu.SemaphoreType.DMA,
    ],
)


@jax.jit
def kernel(x):
    return _write(_reduce(x))


# speedup vs baseline: 1.0696x; 1.0696x over previous
import jax
import jax.numpy as jnp
from jax import lax
from jax.experimental import pallas as pl
from jax.experimental.pallas import tpu as pltpu
from jax.experimental.pallas import tpu_sc as plsc

ROWS, COLS = 16384, 128
NUM_CORES, NUM_SUBCORES = 2, 16
NUM_WORKERS = NUM_CORES * NUM_SUBCORES
ROWS_PER_WORKER = ROWS // NUM_WORKERS
BLK = 64
GRID = 16
BR = ROWS // GRID

_MESH = plsc.VectorSubcoreMesh(
    core_axis_name="c", subcore_axis_name="s",
    num_cores=NUM_CORES, num_subcores=NUM_SUBCORES,
)


def _tc_reduce_body(x_ref, blk_ref, acc_ref):
    i = pl.program_id(0)
    part = jnp.sum(x_ref[...].reshape(BR // 8, 8, COLS), axis=0)

    @pl.when(i == 0)
    def _():
        acc_ref[...] = part

    @pl.when(i > 0)
    def _():
        acc_ref[...] += part

    @pl.when(i == GRID - 1)
    def _():
        s = jnp.sum(acc_ref[...], axis=0, keepdims=True)
        col = lax.broadcasted_iota(jnp.int32, (1, COLS), 1)
        m = jnp.min(jnp.where(s >= 0.0, col, COLS))
        idx = jnp.where(m >= COLS, 0, m)
        blk_ref[...] = jnp.where(
            lax.broadcasted_iota(jnp.int32, (BLK, COLS), 1) == idx, 1.0, -1.0
        ).astype(jnp.float32)


_tc_reduce = pl.pallas_call(
    _tc_reduce_body,
    grid=(GRID,),
    in_specs=[pl.BlockSpec((BR, COLS), lambda i: (i, 0))],
    out_specs=pl.BlockSpec((BLK, COLS), lambda i: (0, 0)),
    out_shape=jax.ShapeDtypeStruct((BLK, COLS), jnp.float32),
    scratch_shapes=[pltpu.VMEM((8, COLS), jnp.float32)],
)


def _sc_write_body(blk_hbm, out_hbm, blk_v, sem):
    cid = lax.axis_index("c")
    sid = lax.axis_index("s")
    wid = cid * NUM_SUBCORES + sid
    base = wid * ROWS_PER_WORKER

    pltpu.sync_copy(blk_hbm, blk_v)
    copies = [
        pltpu.make_async_copy(
            blk_v, out_hbm.at[pl.ds(base + b * BLK, BLK)], sem
        )
        for b in range(ROWS_PER_WORKER // BLK)
    ]
    for c in copies:
        c.start()
    for c in copies:
        c.wait()


_sc_write = pl.kernel(
    _sc_write_body,
    out_type=jax.ShapeDtypeStruct((ROWS, COLS), jnp.float32),
    mesh=_MESH,
    compiler_params=pltpu.CompilerParams(needs_layout_passes=False),
    scratch_types=[
        pltpu.VMEM((BLK, COLS), jnp.float32),
        pltpu.SemaphoreType.DMA,
    ],
)


@jax.jit
def kernel(x):
    return _sc_write(_tc_reduce(x))


# speedup vs baseline: 1.1752x; 1.0987x over previous
import jax
import jax.numpy as jnp
from jax import lax
from jax.experimental import pallas as pl
from jax.experimental.pallas import tpu as pltpu
from jax.experimental.pallas import tpu_sc as plsc

ROWS, COLS = 16384, 128
NUM_CORES, NUM_SUBCORES = 2, 16
NUM_WORKERS = NUM_CORES * NUM_SUBCORES
ROWS_PER_WORKER = ROWS // NUM_WORKERS
BLK = 128
CHUNK = 512
NCHUNK = ROWS // CHUNK
NBUF = 8


def _tc_reduce_body(x_hbm, blk_ref, bufs, *sems):
    for k in range(NBUF):
        pltpu.make_async_copy(
            x_hbm.at[pl.ds(k * CHUNK, CHUNK)], bufs.at[k], sems[k]
        ).start()

    acc = jnp.zeros((1, COLS), jnp.float32)
    for k in range(NCHUNK):
        b = k % NBUF
        pltpu.make_async_copy(
            x_hbm.at[pl.ds(k * CHUNK, CHUNK)], bufs.at[b], sems[b]
        ).wait()
        a = bufs[b].reshape(CHUNK // 8, 8, COLS)
        if k + NBUF < NCHUNK:
            pltpu.make_async_copy(
                x_hbm.at[pl.ds((k + NBUF) * CHUNK, CHUNK)], bufs.at[b], sems[b]
            ).start()
        while a.shape[0] > 1:
            h = a.shape[0] // 2
            a = a[:h] + a[h:]
        acc = acc + jnp.sum(a[0], axis=0, keepdims=True)

    col = lax.broadcasted_iota(jnp.int32, (1, COLS), 1)
    m = jnp.min(jnp.where(acc >= 0.0, col, COLS))
    idx = jnp.where(m >= COLS, 0, m)
    blk_ref[...] = jnp.where(
        lax.broadcasted_iota(jnp.int32, (BLK, COLS), 1) == idx, 1.0, -1.0
    ).astype(jnp.float32)


_tc_reduce = pl.pallas_call(
    _tc_reduce_body,
    in_specs=[pl.BlockSpec(memory_space=pl.MemorySpace.ANY)],
    out_shape=jax.ShapeDtypeStruct((BLK, COLS), jnp.float32),
    scratch_shapes=[pltpu.VMEM((NBUF, CHUNK, COLS), jnp.float32)]
    + [pltpu.SemaphoreType.DMA] * NBUF,
)


_MESH = plsc.VectorSubcoreMesh(
    core_axis_name="c", subcore_axis_name="s",
    num_cores=NUM_CORES, num_subcores=NUM_SUBCORES,
)


def _sc_write_body(blk_hbm, out_hbm, blk_v, sem):
    cid = lax.axis_index("c")
    sid = lax.axis_index("s")
    wid = cid * NUM_SUBCORES + sid
    base = wid * ROWS_PER_WORKER

    pltpu.sync_copy(blk_hbm, blk_v)
    copies = [
        pltpu.make_async_copy(
            blk_v, out_hbm.at[pl.ds(base + b * BLK, BLK)], sem
        )
        for b in range(ROWS_PER_WORKER // BLK)
    ]
    for c in copies:
        c.start()
    for c in copies:
        c.wait()


_sc_write = pl.kernel(
    _sc_write_body,
    out_type=jax.ShapeDtypeStruct((ROWS, COLS), jnp.float32),
    mesh=_MESH,
    compiler_params=pltpu.CompilerParams(needs_layout_passes=False),
    scratch_types=[
        pltpu.VMEM((BLK, COLS), jnp.float32),
        pltpu.SemaphoreType.DMA,
    ],
)


@jax.jit
def kernel(x):
    return _sc_write(_tc_reduce(x))


# speedup vs baseline: 1.2714x; 1.0819x over previous
import jax
import jax.numpy as jnp
from jax import lax
from jax.experimental import pallas as pl
from jax.experimental.pallas import tpu as pltpu
from jax.experimental.pallas import tpu_sc as plsc

ROWS, COLS = 16384, 128
NUM_CORES, NUM_SUBCORES = 2, 16
NUM_WORKERS = NUM_CORES * NUM_SUBCORES
ROWS_PER_WORKER = ROWS // NUM_WORKERS
BLK = 64
CHUNK = 512
NCHUNK = ROWS // CHUNK
NBUF = 8


def _tc_reduce_body(x_hbm, blk_ref, bufs, *sems):
    for k in range(NBUF):
        pltpu.make_async_copy(
            x_hbm.at[pl.ds(k * CHUNK, CHUNK)], bufs.at[k], sems[k]
        ).start()

    acc = jnp.zeros((1, COLS), jnp.float32)
    for k in range(NCHUNK):
        b = k % NBUF
        pltpu.make_async_copy(
            x_hbm.at[pl.ds(k * CHUNK, CHUNK)], bufs.at[b], sems[b]
        ).wait()
        a = bufs[b].reshape(CHUNK // 8, 8, COLS)
        if k + NBUF < NCHUNK:
            pltpu.make_async_copy(
                x_hbm.at[pl.ds((k + NBUF) * CHUNK, CHUNK)], bufs.at[b], sems[b]
            ).start()
        while a.shape[0] > 1:
            h = a.shape[0] // 2
            a = a[:h] + a[h:]
        acc = acc + jnp.sum(a[0], axis=0, keepdims=True)

    col = lax.broadcasted_iota(jnp.int32, (1, COLS), 1)
    m = jnp.min(jnp.where(acc >= 0.0, col, COLS))
    idx = jnp.where(m >= COLS, 0, m)
    blk_ref[...] = jnp.where(
        lax.broadcasted_iota(jnp.int32, (BLK, COLS), 1) == idx, 1.0, -1.0
    ).astype(jnp.float32)


_tc_reduce = pl.pallas_call(
    _tc_reduce_body,
    in_specs=[pl.BlockSpec(memory_space=pl.MemorySpace.ANY)],
    out_shape=jax.ShapeDtypeStruct((BLK, COLS), jnp.float32),
    scratch_shapes=[pltpu.VMEM((NBUF, CHUNK, COLS), jnp.float32)]
    + [pltpu.SemaphoreType.DMA] * NBUF,
)


_MESH = plsc.VectorSubcoreMesh(
    core_axis_name="c", subcore_axis_name="s",
    num_cores=NUM_CORES, num_subcores=NUM_SUBCORES,
)


def _sc_write_body(blk_hbm, out_hbm, blk_v, sem):
    cid = lax.axis_index("c")
    sid = lax.axis_index("s")
    wid = cid * NUM_SUBCORES + sid
    base = wid * ROWS_PER_WORKER

    pltpu.sync_copy(blk_hbm, blk_v)
    copies = [
        pltpu.make_async_copy(
            blk_v, out_hbm.at[pl.ds(base + b * BLK, BLK)], sem
        )
        for b in range(ROWS_PER_WORKER // BLK)
    ]
    for c in copies:
        c.start()
    for c in copies:
        c.wait()


_sc_write = pl.kernel(
    _sc_write_body,
    out_type=jax.ShapeDtypeStruct((ROWS, COLS), jnp.float32),
    mesh=_MESH,
    compiler_params=pltpu.CompilerParams(needs_layout_passes=False),
    scratch_types=[
        pltpu.VMEM((BLK, COLS), jnp.float32),
        pltpu.SemaphoreType.DMA,
    ],
)


@jax.jit
def kernel(x):
    return _sc_write(_tc_reduce(x))


# speedup vs baseline: 1.3438x; 1.0569x over previous
import jax
import jax.numpy as jnp
from jax import lax
from jax.experimental import pallas as pl
from jax.experimental.pallas import tpu as pltpu
from jax.experimental.pallas import tpu_sc as plsc

ROWS, COLS = 16384, 128
NUM_CORES, NUM_SUBCORES = 2, 16
NUM_WORKERS = NUM_CORES * NUM_SUBCORES
ROWS_PER_WORKER = ROWS // NUM_WORKERS
BLK = 64
CHUNK = 512
NCHUNK = ROWS // CHUNK
NBUF = 16


def _tc_reduce_body(x_hbm, blk_ref, bufs, *sems):
    for k in range(NBUF):
        pltpu.make_async_copy(
            x_hbm.at[pl.ds(k * CHUNK, CHUNK)], bufs.at[k], sems[k]
        ).start()

    acc = jnp.zeros((1, COLS), jnp.float32)
    for k in range(NCHUNK):
        b = k % NBUF
        pltpu.make_async_copy(
            x_hbm.at[pl.ds(k * CHUNK, CHUNK)], bufs.at[b], sems[b]
        ).wait()
        a = bufs[b].reshape(CHUNK // 8, 8, COLS)
        if k + NBUF < NCHUNK:
            pltpu.make_async_copy(
                x_hbm.at[pl.ds((k + NBUF) * CHUNK, CHUNK)], bufs.at[b], sems[b]
            ).start()
        while a.shape[0] > 1:
            h = a.shape[0] // 2
            a = a[:h] + a[h:]
        acc = acc + jnp.sum(a[0], axis=0, keepdims=True)

    col = lax.broadcasted_iota(jnp.int32, (1, COLS), 1)
    m = jnp.min(jnp.where(acc >= 0.0, col, COLS))
    idx = jnp.where(m >= COLS, 0, m)
    blk_ref[...] = jnp.where(
        lax.broadcasted_iota(jnp.int32, (BLK, COLS), 1) == idx, 1.0, -1.0
    ).astype(jnp.float32)


_tc_reduce = pl.pallas_call(
    _tc_reduce_body,
    in_specs=[pl.BlockSpec(memory_space=pl.MemorySpace.ANY)],
    out_shape=jax.ShapeDtypeStruct((BLK, COLS), jnp.float32),
    scratch_shapes=[pltpu.VMEM((NBUF, CHUNK, COLS), jnp.float32)]
    + [pltpu.SemaphoreType.DMA] * NBUF,
)


_MESH = plsc.VectorSubcoreMesh(
    core_axis_name="c", subcore_axis_name="s",
    num_cores=NUM_CORES, num_subcores=NUM_SUBCORES,
)


def _sc_write_body(blk_hbm, out_hbm, blk_v, sem):
    cid = lax.axis_index("c")
    sid = lax.axis_index("s")
    wid = cid * NUM_SUBCORES + sid
    base = wid * ROWS_PER_WORKER

    pltpu.sync_copy(blk_hbm, blk_v)
    copies = [
        pltpu.make_async_copy(
            blk_v, out_hbm.at[pl.ds(base + b * BLK, BLK)], sem
        )
        for b in range(ROWS_PER_WORKER // BLK)
    ]
    for c in copies:
        c.start()
    for c in copies:
        c.wait()


_sc_write = pl.kernel(
    _sc_write_body,
    out_type=jax.ShapeDtypeStruct((ROWS, COLS), jnp.float32),
    mesh=_MESH,
    compiler_params=pltpu.CompilerParams(needs_layout_passes=False),
    scratch_types=[
        pltpu.VMEM((BLK, COLS), jnp.float32),
        pltpu.SemaphoreType.DMA,
    ],
)


@jax.jit
def kernel(x):
    return _sc_write(_tc_reduce(x))
